# Initial kernel scaffold; baseline (speedup 1.0000x reference)
import jax
import jax.numpy as jnp
from jax.experimental import pallas as pl


def kernel(mapping, maps):
    raise NotImplementedError("write your pallas kernel here")


# speedup vs baseline: 1.3961x; 1.3961x over previous
import functools

import jax
import jax.numpy as jnp
from jax import lax
from jax.experimental import pallas as pl
from jax.experimental.pallas import tpu as pltpu
from jax.experimental.pallas import tpu_sc as plsc

_DIM = 32
_HEADS = 8
_ROW = _HEADS * _DIM * _DIM

_NC = 2
_NS = 16
_NW = _NC * _NS

_CHUNK = 4


def _sc_body(n_chunks, maps_hbm, idx_hbm, out_hbm, idx_v, buf_a, buf_b,
             gsem_a, gsem_b, osem_a, osem_b):
    wid = lax.axis_index("s") * _NC + lax.axis_index("c")
    base = wid * (n_chunks * _CHUNK)

    pltpu.sync_copy(idx_hbm.at[wid], idx_v)

    def pair(p, carry):
        j0 = 2 * p
        j1 = j0 + 1
        g0 = pltpu.async_copy(maps_hbm.at[idx_v.at[j0]], buf_a, gsem_a)
        g1 = pltpu.async_copy(maps_hbm.at[idx_v.at[j1]], buf_b, gsem_b)
        g0.wait()
        o0 = pltpu.async_copy(
            buf_a, out_hbm.at[pl.ds(base + j0 * _CHUNK, _CHUNK)], osem_a)
        g1.wait()
        o1 = pltpu.async_copy(
            buf_b, out_hbm.at[pl.ds(base + j1 * _CHUNK, _CHUNK)], osem_b)
        o0.wait()
        o1.wait()
        return carry

    lax.fori_loop(0, n_chunks // 2, pair, 0)


@jax.jit
def kernel(mapping, maps):
    batch, seq = mapping.shape
    tokens = batch * seq
    tokens_per_worker = tokens // _NW
    n_chunks = tokens_per_worker // _CHUNK

    idx = mapping.reshape(_NW, n_chunks, _CHUNK)
    table = maps.reshape(maps.shape[0], _ROW // 128, 128)

    mesh = plsc.VectorSubcoreMesh(core_axis_name="c", subcore_axis_name="s")
    run = pl.kernel(
        functools.partial(_sc_body, n_chunks),
        mesh=mesh,
        out_type=jax.ShapeDtypeStruct((tokens, _ROW // 128, 128),
                                      jnp.float32),
        scratch_types=[
            pltpu.VMEM((n_chunks, _CHUNK), jnp.int32),
            pltpu.VMEM((_CHUNK, _ROW // 128, 128), jnp.float32),
            pltpu.VMEM((_CHUNK, _ROW // 128, 128), jnp.float32),
            pltpu.SemaphoreType.DMA,
            pltpu.SemaphoreType.DMA,
            pltpu.SemaphoreType.DMA,
            pltpu.SemaphoreType.DMA,
        ],
    )
    out = run(table, idx)
    return out.reshape(batch, seq, _HEADS, _DIM, _DIM)


# speedup vs baseline: 1.3997x; 1.0026x over previous
import functools

import jax
import jax.numpy as jnp
from jax import lax
from jax.experimental.layout import Format, Layout
from jax.experimental import pallas as pl
from jax.experimental.pallas import tpu as pltpu
from jax.experimental.pallas import tpu_sc as plsc

_DIM = 32
_HEADS = 8
_ROW = _HEADS * _DIM * _DIM

_NC = 2
_NS = 16
_NW = _NC * _NS

_CHUNK = 4


def _sc_body(n_chunks, maps_hbm, idx_hbm, out_hbm, idx_v, buf_a, buf_b,
             gsem_a, gsem_b, osem_a, osem_b):
    wid = lax.axis_index("s") * _NC + lax.axis_index("c")
    base = wid * (n_chunks * _CHUNK)

    pltpu.sync_copy(idx_hbm.at[wid], idx_v)

    def pair(p, carry):
        j0 = 2 * p
        j1 = j0 + 1
        g0 = pltpu.async_copy(maps_hbm.at[idx_v.at[j0]], buf_a, gsem_a)
        g1 = pltpu.async_copy(maps_hbm.at[idx_v.at[j1]], buf_b, gsem_b)
        g0.wait()
        o0 = pltpu.async_copy(
            buf_a, out_hbm.at[pl.ds(base + j0 * _CHUNK, _CHUNK)], osem_a)
        g1.wait()
        o1 = pltpu.async_copy(
            buf_b, out_hbm.at[pl.ds(base + j1 * _CHUNK, _CHUNK)], osem_b)
        o0.wait()
        o1.wait()
        return carry

    lax.fori_loop(0, n_chunks // 2, pair, 0)


def _impl(mapping, maps):
    batch, seq = mapping.shape
    tokens = batch * seq
    tokens_per_worker = tokens // _NW
    n_chunks = tokens_per_worker // _CHUNK

    idx = mapping.reshape(_NW, n_chunks, _CHUNK)
    table = maps.reshape(maps.shape[0], _ROW // 128, 128)

    mesh = plsc.VectorSubcoreMesh(core_axis_name="c", subcore_axis_name="s")
    run = pl.kernel(
        functools.partial(_sc_body, n_chunks),
        mesh=mesh,
        out_type=jax.ShapeDtypeStruct((tokens, _ROW // 128, 128),
                                      jnp.float32),
        scratch_types=[
            pltpu.VMEM((n_chunks, _CHUNK), jnp.int32),
            pltpu.VMEM((_CHUNK, _ROW // 128, 128), jnp.float32),
            pltpu.VMEM((_CHUNK, _ROW // 128, 128), jnp.float32),
            pltpu.SemaphoreType.DMA,
            pltpu.SemaphoreType.DMA,
            pltpu.SemaphoreType.DMA,
            pltpu.SemaphoreType.DMA,
        ],
    )
    out = run(table, idx)
    return out.reshape(batch, seq, _HEADS, _DIM, _DIM)


_jitted = None


def kernel(mapping, maps):
    global _jitted
    if _jitted is None:
        dev = jax.devices()[0]
        fmt = Format(Layout(major_to_minor=(0, 1, 2, 3, 4)),
                     jax.sharding.SingleDeviceSharding(dev))
        _jitted = jax.jit(_impl, out_shardings=fmt)
    return _jitted(mapping, maps)


# speedup vs baseline: 1.7069x; 1.2194x over previous
import functools

import jax
import jax.numpy as jnp
from jax import lax
from jax.experimental import pallas as pl
from jax.experimental.pallas import tpu as pltpu
from jax.experimental.pallas import tpu_sc as plsc

_DIM = 32
_HEADS = 8
_ROW = _HEADS * _DIM * _DIM

_NC = 2
_NS = 16
_NW = _NC * _NS

_CHUNK = 4

_SBLK = 512
_EBLK = 512


def _sc_body(n_chunks, maps_hbm, idx_hbm, out_hbm, idx_v, buf_a, buf_b,
             gsem_a, gsem_b, osem_a, osem_b):
    wid = lax.axis_index("s") * _NC + lax.axis_index("c")
    base = wid * (n_chunks * _CHUNK)

    pltpu.sync_copy(idx_hbm.at[wid], idx_v)

    def pair(p, carry):
        j0 = 2 * p
        j1 = j0 + 1
        g0 = pltpu.async_copy(maps_hbm.at[idx_v.at[j0]], buf_a, gsem_a)
        g1 = pltpu.async_copy(maps_hbm.at[idx_v.at[j1]], buf_b, gsem_b)
        g0.wait()
        o0 = pltpu.async_copy(
            buf_a, out_hbm.at[pl.ds(base + j0 * _CHUNK, _CHUNK)], osem_a)
        g1.wait()
        o1 = pltpu.async_copy(
            buf_b, out_hbm.at[pl.ds(base + j1 * _CHUNK, _CHUNK)], osem_b)
        o0.wait()
        o1.wait()
        return carry

    lax.fori_loop(0, n_chunks // 2, pair, 0)


def _sc_gather(table, idx, tokens):
    mesh = plsc.VectorSubcoreMesh(core_axis_name="c", subcore_axis_name="s")
    n_chunks = idx.shape[1]
    run = pl.kernel(
        functools.partial(_sc_body, n_chunks),
        mesh=mesh,
        out_type=jax.ShapeDtypeStruct((tokens, _ROW // 128, 128),
                                      jnp.float32),
        scratch_types=[
            pltpu.VMEM((n_chunks, _CHUNK), jnp.int32),
            pltpu.VMEM((_CHUNK, _ROW // 128, 128), jnp.float32),
            pltpu.VMEM((_CHUNK, _ROW // 128, 128), jnp.float32),
            pltpu.SemaphoreType.DMA,
            pltpu.SemaphoreType.DMA,
            pltpu.SemaphoreType.DMA,
            pltpu.SemaphoreType.DMA,
        ],
    )
    return run(table, idx)


def _transpose_body(in_ref, out_ref):
    out_ref[0] = jnp.swapaxes(in_ref[0], 0, 1)


def _tc_transpose(x):
    batch, s, e = x.shape
    grid = (batch, s // _SBLK, e // _EBLK)
    return pl.pallas_call(
        _transpose_body,
        grid=grid,
        in_specs=[pl.BlockSpec((1, _SBLK, _EBLK),
                               lambda b, i, j: (b, i, j))],
        out_specs=pl.BlockSpec((1, _EBLK, _SBLK),
                               lambda b, i, j: (b, j, i)),
        out_shape=jax.ShapeDtypeStruct((batch, e, s), jnp.float32),
    )(x)


@jax.jit
def kernel(mapping, maps):
    batch, seq = mapping.shape
    tokens = batch * seq
    tokens_per_worker = tokens // _NW
    n_chunks = tokens_per_worker // _CHUNK

    idx = mapping.reshape(_NW, n_chunks, _CHUNK)
    table = maps.reshape(maps.shape[0], _ROW // 128, 128)

    gathered = _sc_gather(table, idx, tokens)
    gathered = gathered.reshape(batch, seq, _ROW)
    out_phys = _tc_transpose(gathered)
    out = out_phys.reshape(batch, _HEADS, _DIM, _DIM, seq)
    return jnp.transpose(out, (0, 4, 1, 2, 3))


# speedup vs baseline: 2.7571x; 1.6153x over previous
import functools

import jax
import jax.numpy as jnp
from jax import lax
from jax.experimental import pallas as pl
from jax.experimental.pallas import tpu as pltpu
from jax.experimental.pallas import tpu_sc as plsc

_DIM = 32
_HEADS = 8
_ROW = _HEADS * _DIM * _DIM

_NC = 2
_NS = 16
_NW = _NC * _NS

_CHUNK = 4

_SBLK = 512
_ECHB = 8


def _sc_body(n_chunks, maps_hbm, idx_hbm, out_hbm, idx_v, buf_a, buf_b,
             gsem_a, gsem_b, osem_a, osem_b):
    wid = lax.axis_index("s") * _NC + lax.axis_index("c")
    base = wid * (n_chunks * _CHUNK)

    pltpu.sync_copy(idx_hbm.at[wid], idx_v)

    def pair(p, carry):
        j0 = 2 * p
        j1 = j0 + 1
        g0 = pltpu.async_copy(maps_hbm.at[idx_v.at[j0]], buf_a, gsem_a)
        g1 = pltpu.async_copy(maps_hbm.at[idx_v.at[j1]], buf_b, gsem_b)
        g0.wait()
        o0 = pltpu.async_copy(
            buf_a, out_hbm.at[pl.ds(base + j0 * _CHUNK, _CHUNK)], osem_a)
        g1.wait()
        o1 = pltpu.async_copy(
            buf_b, out_hbm.at[pl.ds(base + j1 * _CHUNK, _CHUNK)], osem_b)
        o0.wait()
        o1.wait()
        return carry

    lax.fori_loop(0, n_chunks // 2, pair, 0)


def _sc_gather(table, idx, tokens):
    mesh = plsc.VectorSubcoreMesh(core_axis_name="c", subcore_axis_name="s")
    n_chunks = idx.shape[1]
    run = pl.kernel(
        functools.partial(_sc_body, n_chunks),
        mesh=mesh,
        out_type=jax.ShapeDtypeStruct((tokens, _ROW // 128, 128),
                                      jnp.float32),
        scratch_types=[
            pltpu.VMEM((n_chunks, _CHUNK), jnp.int32),
            pltpu.VMEM((_CHUNK, _ROW // 128, 128), jnp.float32),
            pltpu.VMEM((_CHUNK, _ROW // 128, 128), jnp.float32),
            pltpu.SemaphoreType.DMA,
            pltpu.SemaphoreType.DMA,
            pltpu.SemaphoreType.DMA,
            pltpu.SemaphoreType.DMA,
        ],
    )
    return run(table, idx)


def _transpose_body(in_ref, out_ref):
    for ec in range(_ECHB):
        out_ref[0, ec] = jnp.swapaxes(in_ref[0, :, ec, :], 0, 1)


def _tc_transpose(x):
    batch, seq, ecs, lanes = x.shape
    grid = (batch, seq // _SBLK, ecs // _ECHB)
    return pl.pallas_call(
        _transpose_body,
        grid=grid,
        in_specs=[pl.BlockSpec((1, _SBLK, _ECHB, lanes),
                               lambda b, i, j: (b, i, j, 0))],
        out_specs=pl.BlockSpec((1, _ECHB, lanes, _SBLK),
                               lambda b, i, j: (b, j, 0, i)),
        out_shape=jax.ShapeDtypeStruct((batch, ecs, lanes, seq),
                                       jnp.float32),
    )(x)


@jax.jit
def kernel(mapping, maps):
    batch, seq = mapping.shape
    tokens = batch * seq
    tokens_per_worker = tokens // _NW
    n_chunks = tokens_per_worker // _CHUNK

    idx = mapping.reshape(_NW, n_chunks, _CHUNK)
    table = maps.reshape(maps.shape[0], _ROW // 128, 128)

    gathered = _sc_gather(table, idx, tokens)
    gathered = gathered.reshape(batch, seq, _ROW // 128, 128)
    out_phys = _tc_transpose(gathered)
    out = out_phys.reshape(batch, _HEADS, _DIM, _DIM, seq)
    return jnp.transpose(out, (0, 4, 1, 2, 3))


# speedup vs baseline: 3.0207x; 1.0956x over previous
import functools

import jax
import jax.numpy as jnp
from jax import lax
from jax.experimental import pallas as pl
from jax.experimental.pallas import tpu as pltpu
from jax.experimental.pallas import tpu_sc as plsc

_DIM = 32
_HEADS = 8
_ROW = _HEADS * _DIM * _DIM

_NC = 2
_NS = 16
_NW = _NC * _NS

_CHUNK = 4

_SBLK = 512
_ECHB = 8


def _sc_body(n_chunks, maps_hbm, idx_hbm, out_hbm, idx_v, buf_a, buf_b,
             gsem_a, gsem_b, osem_a, osem_b):
    wid = lax.axis_index("s") * _NC + lax.axis_index("c")
    base = wid * (n_chunks * _CHUNK)

    pltpu.sync_copy(idx_hbm.at[wid], idx_v)

    def pair(p, carry):
        j0 = 2 * p
        j1 = j0 + 1
        g0 = pltpu.async_copy(maps_hbm.at[idx_v.at[j0]], buf_a, gsem_a)
        g1 = pltpu.async_copy(maps_hbm.at[idx_v.at[j1]], buf_b, gsem_b)
        g0.wait()
        o0 = pltpu.async_copy(
            buf_a, out_hbm.at[pl.ds(base + j0 * _CHUNK, _CHUNK)], osem_a)
        g1.wait()
        o1 = pltpu.async_copy(
            buf_b, out_hbm.at[pl.ds(base + j1 * _CHUNK, _CHUNK)], osem_b)
        o0.wait()
        o1.wait()
        return carry

    lax.fori_loop(0, n_chunks // 2, pair, 0)


def _sc_gather(table, idx, tokens):
    mesh = plsc.VectorSubcoreMesh(core_axis_name="c", subcore_axis_name="s")
    n_chunks = idx.shape[1]
    run = pl.kernel(
        functools.partial(_sc_body, n_chunks),
        mesh=mesh,
        out_type=jax.ShapeDtypeStruct((tokens, _ROW // 128, 128),
                                      jnp.float32),
        scratch_types=[
            pltpu.VMEM((n_chunks, _CHUNK), jnp.int32),
            pltpu.VMEM((_CHUNK, _ROW // 128, 128), jnp.float32),
            pltpu.VMEM((_CHUNK, _ROW // 128, 128), jnp.float32),
            pltpu.SemaphoreType.DMA,
            pltpu.SemaphoreType.DMA,
            pltpu.SemaphoreType.DMA,
            pltpu.SemaphoreType.DMA,
        ],
    )
    return run(table, idx)


def _transpose_body(in_ref, out_ref):
    for ec in range(_ECHB):
        out_ref[0, ec] = jnp.swapaxes(in_ref[0, :, ec, :], 0, 1)


def _transpose_part_body(in_ref, *rest):
    out_ref = rest[-1]
    for ec in range(_ECHB):
        out_ref[0, ec] = jnp.swapaxes(in_ref[0, :, ec, :], 0, 1)


def _tc_transpose_part(part, b, buf, out_shape):
    one, seq, ecs, lanes = part.shape
    grid = (seq // _SBLK, ecs // _ECHB)
    in_specs = [
        pl.BlockSpec((1, _SBLK, _ECHB, lanes), lambda i, j: (0, i, j, 0)),
    ]
    kwargs = {}
    operands = (part,)
    if buf is not None:
        in_specs.append(pl.BlockSpec(memory_space=pl.ANY))
        operands = (part, buf)
        kwargs["input_output_aliases"] = {1: 0}
    return pl.pallas_call(
        _transpose_part_body,
        grid=grid,
        in_specs=in_specs,
        out_specs=pl.BlockSpec((1, _ECHB, lanes, _SBLK),
                               lambda i, j: (b, j, 0, i)),
        out_shape=out_shape,
        **kwargs,
    )(*operands)


def _tc_transpose(x):
    batch, seq, ecs, lanes = x.shape
    grid = (batch, seq // _SBLK, ecs // _ECHB)
    return pl.pallas_call(
        _transpose_body,
        grid=grid,
        in_specs=[pl.BlockSpec((1, _SBLK, _ECHB, lanes),
                               lambda b, i, j: (b, i, j, 0))],
        out_specs=pl.BlockSpec((1, _ECHB, lanes, _SBLK),
                               lambda b, i, j: (b, j, 0, i)),
        out_shape=jax.ShapeDtypeStruct((batch, ecs, lanes, seq),
                                       jnp.float32),
    )(x)


@jax.jit
def kernel(mapping, maps):
    batch, seq = mapping.shape
    tokens = batch * seq
    tokens_per_worker = tokens // _NW
    n_chunks = tokens_per_worker // _CHUNK

    idx = mapping.reshape(_NW, n_chunks, _CHUNK)
    table = maps.reshape(maps.shape[0], _ROW // 128, 128)

    out_shape = jax.ShapeDtypeStruct((batch, _ROW // 128, 128, seq),
                                     jnp.float32)
    buf = None
    for b in range(batch):
        idx_b = mapping[b].reshape(_NW, seq // _NW // _CHUNK, _CHUNK)
        g = _sc_gather(table, idx_b, seq)
        g = g.reshape(1, seq, _ROW // 128, 128)
        buf = _tc_transpose_part(g, b, buf, out_shape)
    out = buf.reshape(batch, _HEADS, _DIM, _DIM, seq)
    return jnp.transpose(out, (0, 4, 1, 2, 3))


# speedup vs baseline: 3.0637x; 1.0142x over previous
import functools

import jax
import jax.numpy as jnp
from jax import lax
from jax.experimental import pallas as pl
from jax.experimental.pallas import tpu as pltpu
from jax.experimental.pallas import tpu_sc as plsc

_DIM = 32
_HEADS = 8
_ROW = _HEADS * _DIM * _DIM

_NC = 2
_NS = 16
_NW = _NC * _NS

_CHUNK = 4

_SBLK = 512
_ECHB = 8


def _sc_body(n_chunks, maps_hbm, idx_hbm, out_hbm, idx_v, buf_a, buf_b,
             gsem_a, gsem_b, osem_a, osem_b):
    wid = lax.axis_index("s") * _NC + lax.axis_index("c")
    base = wid * (n_chunks * _CHUNK)

    pltpu.sync_copy(idx_hbm.at[wid], idx_v)

    def pair(p, carry):
        j0 = 2 * p
        j1 = j0 + 1
        g0 = pltpu.async_copy(maps_hbm.at[idx_v.at[j0]], buf_a, gsem_a)
        g1 = pltpu.async_copy(maps_hbm.at[idx_v.at[j1]], buf_b, gsem_b)
        g0.wait()
        o0 = pltpu.async_copy(
            buf_a, out_hbm.at[pl.ds(base + j0 * _CHUNK, _CHUNK)], osem_a)
        g1.wait()
        o1 = pltpu.async_copy(
            buf_b, out_hbm.at[pl.ds(base + j1 * _CHUNK, _CHUNK)], osem_b)
        o0.wait()
        o1.wait()
        return carry

    lax.fori_loop(0, n_chunks // 2, pair, 0)


def _sc_gather(table, idx, tokens):
    mesh = plsc.VectorSubcoreMesh(core_axis_name="c", subcore_axis_name="s")
    n_chunks = idx.shape[1]
    run = pl.kernel(
        functools.partial(_sc_body, n_chunks),
        mesh=mesh,
        out_type=jax.ShapeDtypeStruct((tokens, _ROW // 128, 128),
                                      jnp.float32),
        scratch_types=[
            pltpu.VMEM((n_chunks, _CHUNK), jnp.int32),
            pltpu.VMEM((_CHUNK, _ROW // 128, 128), jnp.float32),
            pltpu.VMEM((_CHUNK, _ROW // 128, 128), jnp.float32),
            pltpu.SemaphoreType.DMA,
            pltpu.SemaphoreType.DMA,
            pltpu.SemaphoreType.DMA,
            pltpu.SemaphoreType.DMA,
        ],
    )
    return run(table, idx)


def _transpose_in_body(in_ref, out_ref):
    for eb in range(_ECHB):
        out_ref[:, eb, :] = jnp.swapaxes(in_ref[eb], 0, 1)


def _tc_transpose_in(m4):
    ecs, lanes, npos = m4.shape
    grid = (npos // _SBLK, ecs // _ECHB)
    return pl.pallas_call(
        _transpose_in_body,
        grid=grid,
        in_specs=[pl.BlockSpec((_ECHB, lanes, _SBLK),
                               lambda i, j: (j, 0, i))],
        out_specs=pl.BlockSpec((_SBLK, _ECHB, lanes),
                               lambda i, j: (i, j, 0)),
        out_shape=jax.ShapeDtypeStruct((npos, ecs, lanes), jnp.float32),
    )(m4)


def _transpose_body(in_ref, out_ref):
    for ec in range(_ECHB):
        out_ref[0, ec] = jnp.swapaxes(in_ref[0, :, ec, :], 0, 1)


def _transpose_part_body(in_ref, *rest):
    out_ref = rest[-1]
    for ec in range(_ECHB):
        out_ref[0, ec] = jnp.swapaxes(in_ref[0, :, ec, :], 0, 1)


def _tc_transpose_part(part, b, buf, out_shape):
    one, seq, ecs, lanes = part.shape
    grid = (seq // _SBLK, ecs // _ECHB)
    in_specs = [
        pl.BlockSpec((1, _SBLK, _ECHB, lanes), lambda i, j: (0, i, j, 0)),
    ]
    kwargs = {}
    operands = (part,)
    if buf is not None:
        in_specs.append(pl.BlockSpec(memory_space=pl.ANY))
        operands = (part, buf)
        kwargs["input_output_aliases"] = {1: 0}
    return pl.pallas_call(
        _transpose_part_body,
        grid=grid,
        in_specs=in_specs,
        out_specs=pl.BlockSpec((1, _ECHB, lanes, _SBLK),
                               lambda i, j: (b, j, 0, i)),
        out_shape=out_shape,
        **kwargs,
    )(*operands)


def _tc_transpose(x):
    batch, seq, ecs, lanes = x.shape
    grid = (batch, seq // _SBLK, ecs // _ECHB)
    return pl.pallas_call(
        _transpose_body,
        grid=grid,
        in_specs=[pl.BlockSpec((1, _SBLK, _ECHB, lanes),
                               lambda b, i, j: (b, i, j, 0))],
        out_specs=pl.BlockSpec((1, _ECHB, lanes, _SBLK),
                               lambda b, i, j: (b, j, 0, i)),
        out_shape=jax.ShapeDtypeStruct((batch, ecs, lanes, seq),
                                       jnp.float32),
    )(x)


@jax.jit
def kernel(mapping, maps):
    batch, seq = mapping.shape
    tokens = batch * seq
    tokens_per_worker = tokens // _NW
    n_chunks = tokens_per_worker // _CHUNK

    m4 = jnp.transpose(maps, (1, 2, 3, 0)).reshape(_ROW // 128, 128,
                                                   maps.shape[0])
    table = _tc_transpose_in(m4)

    out_shape = jax.ShapeDtypeStruct((batch, _ROW // 128, 128, seq),
                                     jnp.float32)
    buf = None
    for b in range(batch):
        idx_b = mapping[b].reshape(_NW, seq // _NW // _CHUNK, _CHUNK)
        g = _sc_gather(table, idx_b, seq)
        g = g.reshape(1, seq, _ROW // 128, 128)
        buf = _tc_transpose_part(g, b, buf, out_shape)
    out = buf.reshape(batch, _HEADS, _DIM, _DIM, seq)
    return jnp.transpose(out, (0, 4, 1, 2, 3))


# speedup vs baseline: 3.1291x; 1.0214x over previous
import functools

import jax
import jax.numpy as jnp
from jax import lax
from jax.experimental import pallas as pl
from jax.experimental.pallas import tpu as pltpu
from jax.experimental.pallas import tpu_sc as plsc

_DIM = 32
_HEADS = 8
_ROW = _HEADS * _DIM * _DIM

_NC = 2
_NS = 16
_NW = _NC * _NS

_CHUNK = 4

_SBLK = 1024
_ECHB = 8


def _sc_body(n_chunks, maps_hbm, idx_hbm, out_hbm, idx_v, buf_a, buf_b,
             gsem_a, gsem_b, osem_a, osem_b):
    wid = lax.axis_index("s") * _NC + lax.axis_index("c")
    base = wid * (n_chunks * _CHUNK)

    pltpu.sync_copy(idx_hbm.at[wid], idx_v)

    def pair(p, carry):
        j0 = 2 * p
        j1 = j0 + 1
        g0 = pltpu.async_copy(maps_hbm.at[idx_v.at[j0]], buf_a, gsem_a)
        g1 = pltpu.async_copy(maps_hbm.at[idx_v.at[j1]], buf_b, gsem_b)
        g0.wait()
        o0 = pltpu.async_copy(
            buf_a, out_hbm.at[pl.ds(base + j0 * _CHUNK, _CHUNK)], osem_a)
        g1.wait()
        o1 = pltpu.async_copy(
            buf_b, out_hbm.at[pl.ds(base + j1 * _CHUNK, _CHUNK)], osem_b)
        o0.wait()
        o1.wait()
        return carry

    lax.fori_loop(0, n_chunks // 2, pair, 0)


def _sc_gather(table, idx, tokens):
    mesh = plsc.VectorSubcoreMesh(core_axis_name="c", subcore_axis_name="s")
    n_chunks = idx.shape[1]
    run = pl.kernel(
        functools.partial(_sc_body, n_chunks),
        mesh=mesh,
        out_type=jax.ShapeDtypeStruct((tokens, _ROW // 128, 128),
                                      jnp.float32),
        scratch_types=[
            pltpu.VMEM((n_chunks, _CHUNK), jnp.int32),
            pltpu.VMEM((_CHUNK, _ROW // 128, 128), jnp.float32),
            pltpu.VMEM((_CHUNK, _ROW // 128, 128), jnp.float32),
            pltpu.SemaphoreType.DMA,
            pltpu.SemaphoreType.DMA,
            pltpu.SemaphoreType.DMA,
            pltpu.SemaphoreType.DMA,
        ],
    )
    return run(table, idx)


def _transpose_in_body(in_ref, out_ref):
    for eb in range(_ECHB):
        out_ref[:, eb, :] = jnp.swapaxes(in_ref[eb], 0, 1)


def _tc_transpose_in(m4):
    ecs, lanes, npos = m4.shape
    grid = (npos // _SBLK, ecs // _ECHB)
    return pl.pallas_call(
        _transpose_in_body,
        grid=grid,
        in_specs=[pl.BlockSpec((_ECHB, lanes, _SBLK),
                               lambda i, j: (j, 0, i))],
        out_specs=pl.BlockSpec((_SBLK, _ECHB, lanes),
                               lambda i, j: (i, j, 0)),
        out_shape=jax.ShapeDtypeStruct((npos, ecs, lanes), jnp.float32),
    )(m4)


def _transpose_body(in_ref, out_ref):
    for ec in range(_ECHB):
        out_ref[0, ec] = jnp.swapaxes(in_ref[0, :, ec, :], 0, 1)


def _transpose_part_body(in_ref, *rest):
    out_ref = rest[-1]
    for ec in range(_ECHB):
        out_ref[0, ec] = jnp.swapaxes(in_ref[0, :, ec, :], 0, 1)


def _tc_transpose_part(part, b, buf, out_shape):
    one, seq, ecs, lanes = part.shape
    grid = (seq // _SBLK, ecs // _ECHB)
    in_specs = [
        pl.BlockSpec((1, _SBLK, _ECHB, lanes), lambda i, j: (0, i, j, 0)),
    ]
    kwargs = {}
    operands = (part,)
    if buf is not None:
        in_specs.append(pl.BlockSpec(memory_space=pl.ANY))
        operands = (part, buf)
        kwargs["input_output_aliases"] = {1: 0}
    return pl.pallas_call(
        _transpose_part_body,
        grid=grid,
        in_specs=in_specs,
        out_specs=pl.BlockSpec((1, _ECHB, lanes, _SBLK),
                               lambda i, j: (b, j, 0, i)),
        out_shape=out_shape,
        **kwargs,
    )(*operands)


def _tc_transpose(x):
    batch, seq, ecs, lanes = x.shape
    grid = (batch, seq // _SBLK, ecs // _ECHB)
    return pl.pallas_call(
        _transpose_body,
        grid=grid,
        in_specs=[pl.BlockSpec((1, _SBLK, _ECHB, lanes),
                               lambda b, i, j: (b, i, j, 0))],
        out_specs=pl.BlockSpec((1, _ECHB, lanes, _SBLK),
                               lambda b, i, j: (b, j, 0, i)),
        out_shape=jax.ShapeDtypeStruct((batch, ecs, lanes, seq),
                                       jnp.float32),
    )(x)


@jax.jit
def kernel(mapping, maps):
    batch, seq = mapping.shape
    tokens = batch * seq
    tokens_per_worker = tokens // _NW
    n_chunks = tokens_per_worker // _CHUNK

    m4 = jnp.transpose(maps, (1, 2, 3, 0)).reshape(_ROW // 128, 128,
                                                   maps.shape[0])
    table = _tc_transpose_in(m4)

    out_shape = jax.ShapeDtypeStruct((batch, _ROW // 128, 128, seq),
                                     jnp.float32)
    buf = None
    for b in range(batch):
        idx_b = mapping[b].reshape(_NW, seq // _NW // _CHUNK, _CHUNK)
        g = _sc_gather(table, idx_b, seq)
        g = g.reshape(1, seq, _ROW // 128, 128)
        buf = _tc_transpose_part(g, b, buf, out_shape)
    out = buf.reshape(batch, _HEADS, _DIM, _DIM, seq)
    return jnp.transpose(out, (0, 4, 1, 2, 3))


# speedup vs baseline: 3.1776x; 1.0155x over previous
import functools

import jax
import jax.numpy as jnp
from jax import lax
from jax.experimental import pallas as pl
from jax.experimental.pallas import tpu as pltpu
from jax.experimental.pallas import tpu_sc as plsc

_DIM = 32
_HEADS = 8
_ROW = _HEADS * _DIM * _DIM

_NC = 2
_NS = 16
_NW = _NC * _NS

_CHUNK = 4

_SBLK = 1024
_ECHB = 16


def _sc_body(n_chunks, maps_hbm, idx_hbm, out_hbm, idx_v, buf_a, buf_b,
             gsem_a, gsem_b, osem_a, osem_b):
    wid = lax.axis_index("s") * _NC + lax.axis_index("c")
    base = wid * (n_chunks * _CHUNK)

    pltpu.sync_copy(idx_hbm.at[wid], idx_v)

    def pair(p, carry):
        j0 = 2 * p
        j1 = j0 + 1
        g0 = pltpu.async_copy(maps_hbm.at[idx_v.at[j0]], buf_a, gsem_a)
        g1 = pltpu.async_copy(maps_hbm.at[idx_v.at[j1]], buf_b, gsem_b)
        g0.wait()
        o0 = pltpu.async_copy(
            buf_a, out_hbm.at[pl.ds(base + j0 * _CHUNK, _CHUNK)], osem_a)
        g1.wait()
        o1 = pltpu.async_copy(
            buf_b, out_hbm.at[pl.ds(base + j1 * _CHUNK, _CHUNK)], osem_b)
        o0.wait()
        o1.wait()
        return carry

    lax.fori_loop(0, n_chunks // 2, pair, 0)


def _sc_gather(table, idx, tokens):
    mesh = plsc.VectorSubcoreMesh(core_axis_name="c", subcore_axis_name="s")
    n_chunks = idx.shape[1]
    run = pl.kernel(
        functools.partial(_sc_body, n_chunks),
        mesh=mesh,
        out_type=jax.ShapeDtypeStruct((tokens, _ROW // 128, 128),
                                      jnp.float32),
        scratch_types=[
            pltpu.VMEM((n_chunks, _CHUNK), jnp.int32),
            pltpu.VMEM((_CHUNK, _ROW // 128, 128), jnp.float32),
            pltpu.VMEM((_CHUNK, _ROW // 128, 128), jnp.float32),
            pltpu.SemaphoreType.DMA,
            pltpu.SemaphoreType.DMA,
            pltpu.SemaphoreType.DMA,
            pltpu.SemaphoreType.DMA,
        ],
    )
    return run(table, idx)


def _transpose_in_body(in_ref, out_ref):
    for eb in range(_ECHB):
        out_ref[:, eb, :] = jnp.swapaxes(in_ref[eb], 0, 1)


def _tc_transpose_in(m4):
    ecs, lanes, npos = m4.shape
    grid = (npos // _SBLK, ecs // _ECHB)
    return pl.pallas_call(
        _transpose_in_body,
        grid=grid,
        in_specs=[pl.BlockSpec((_ECHB, lanes, _SBLK),
                               lambda i, j: (j, 0, i))],
        out_specs=pl.BlockSpec((_SBLK, _ECHB, lanes),
                               lambda i, j: (i, j, 0)),
        out_shape=jax.ShapeDtypeStruct((npos, ecs, lanes), jnp.float32),
    )(m4)


def _transpose_body(in_ref, out_ref):
    for ec in range(_ECHB):
        out_ref[0, ec] = jnp.swapaxes(in_ref[0, :, ec, :], 0, 1)


def _transpose_part_body(in_ref, *rest):
    out_ref = rest[-1]
    for ec in range(_ECHB):
        out_ref[0, ec] = jnp.swapaxes(in_ref[0, :, ec, :], 0, 1)


def _tc_transpose_part(part, b, buf, out_shape):
    one, seq, ecs, lanes = part.shape
    grid = (seq // _SBLK, ecs // _ECHB)
    in_specs = [
        pl.BlockSpec((1, _SBLK, _ECHB, lanes), lambda i, j: (0, i, j, 0)),
    ]
    kwargs = {}
    operands = (part,)
    if buf is not None:
        in_specs.append(pl.BlockSpec(memory_space=pl.ANY))
        operands = (part, buf)
        kwargs["input_output_aliases"] = {1: 0}
    return pl.pallas_call(
        _transpose_part_body,
        grid=grid,
        in_specs=in_specs,
        out_specs=pl.BlockSpec((1, _ECHB, lanes, _SBLK),
                               lambda i, j: (b, j, 0, i)),
        out_shape=out_shape,
        **kwargs,
    )(*operands)


def _tc_transpose(x):
    batch, seq, ecs, lanes = x.shape
    grid = (batch, seq // _SBLK, ecs // _ECHB)
    return pl.pallas_call(
        _transpose_body,
        grid=grid,
        in_specs=[pl.BlockSpec((1, _SBLK, _ECHB, lanes),
                               lambda b, i, j: (b, i, j, 0))],
        out_specs=pl.BlockSpec((1, _ECHB, lanes, _SBLK),
                               lambda b, i, j: (b, j, 0, i)),
        out_shape=jax.ShapeDtypeStruct((batch, ecs, lanes, seq),
                                       jnp.float32),
    )(x)


@jax.jit
def kernel(mapping, maps):
    batch, seq = mapping.shape
    tokens = batch * seq
    tokens_per_worker = tokens // _NW
    n_chunks = tokens_per_worker // _CHUNK

    m4 = jnp.transpose(maps, (1, 2, 3, 0)).reshape(_ROW // 128, 128,
                                                   maps.shape[0])
    table = _tc_transpose_in(m4)

    out_shape = jax.ShapeDtypeStruct((batch, _ROW // 128, 128, seq),
                                     jnp.float32)
    buf = None
    for b in range(batch):
        idx_b = mapping[b].reshape(_NW, seq // _NW // _CHUNK, _CHUNK)
        g = _sc_gather(table, idx_b, seq)
        g = g.reshape(1, seq, _ROW // 128, 128)
        buf = _tc_transpose_part(g, b, buf, out_shape)
    out = buf.reshape(batch, _HEADS, _DIM, _DIM, seq)
    return jnp.transpose(out, (0, 4, 1, 2, 3))


# speedup vs baseline: 3.1925x; 1.0047x over previous
import functools

import jax
import jax.numpy as jnp
from jax import lax
from jax.experimental import pallas as pl
from jax.experimental.pallas import tpu as pltpu
from jax.experimental.pallas import tpu_sc as plsc

_DIM = 32
_HEADS = 8
_ROW = _HEADS * _DIM * _DIM

_NC = 2
_NS = 16
_NW = _NC * _NS

_CHUNK = 4

_SBLK = 2048
_ECHB = 8


def _sc_body(n_chunks, maps_hbm, idx_hbm, out_hbm, idx_v, buf_a, buf_b,
             gsem_a, gsem_b, osem_a, osem_b):
    wid = lax.axis_index("s") * _NC + lax.axis_index("c")
    base = wid * (n_chunks * _CHUNK)

    pltpu.sync_copy(idx_hbm.at[wid], idx_v)

    def pair(p, carry):
        j0 = 2 * p
        j1 = j0 + 1
        g0 = pltpu.async_copy(maps_hbm.at[idx_v.at[j0]], buf_a, gsem_a)
        g1 = pltpu.async_copy(maps_hbm.at[idx_v.at[j1]], buf_b, gsem_b)
        g0.wait()
        o0 = pltpu.async_copy(
            buf_a, out_hbm.at[pl.ds(base + j0 * _CHUNK, _CHUNK)], osem_a)
        g1.wait()
        o1 = pltpu.async_copy(
            buf_b, out_hbm.at[pl.ds(base + j1 * _CHUNK, _CHUNK)], osem_b)
        o0.wait()
        o1.wait()
        return carry

    lax.fori_loop(0, n_chunks // 2, pair, 0)


def _sc_gather(table, idx, tokens):
    mesh = plsc.VectorSubcoreMesh(core_axis_name="c", subcore_axis_name="s")
    n_chunks = idx.shape[1]
    run = pl.kernel(
        functools.partial(_sc_body, n_chunks),
        mesh=mesh,
        out_type=jax.ShapeDtypeStruct((tokens, _ROW // 128, 128),
                                      jnp.float32),
        scratch_types=[
            pltpu.VMEM((n_chunks, _CHUNK), jnp.int32),
            pltpu.VMEM((_CHUNK, _ROW // 128, 128), jnp.float32),
            pltpu.VMEM((_CHUNK, _ROW // 128, 128), jnp.float32),
            pltpu.SemaphoreType.DMA,
            pltpu.SemaphoreType.DMA,
            pltpu.SemaphoreType.DMA,
            pltpu.SemaphoreType.DMA,
        ],
    )
    return run(table, idx)


def _transpose_in_body(in_ref, out_ref):
    for eb in range(_ECHB):
        out_ref[:, eb, :] = jnp.swapaxes(in_ref[eb], 0, 1)


def _tc_transpose_in(m4):
    ecs, lanes, npos = m4.shape
    grid = (npos // _SBLK, ecs // _ECHB)
    return pl.pallas_call(
        _transpose_in_body,
        grid=grid,
        in_specs=[pl.BlockSpec((_ECHB, lanes, _SBLK),
                               lambda i, j: (j, 0, i))],
        out_specs=pl.BlockSpec((_SBLK, _ECHB, lanes),
                               lambda i, j: (i, j, 0)),
        out_shape=jax.ShapeDtypeStruct((npos, ecs, lanes), jnp.float32),
    )(m4)


def _transpose_body(in_ref, out_ref):
    for ec in range(_ECHB):
        out_ref[0, ec] = jnp.swapaxes(in_ref[0, :, ec, :], 0, 1)


def _transpose_part_body(in_ref, *rest):
    out_ref = rest[-1]
    for ec in range(_ECHB):
        out_ref[0, ec] = jnp.swapaxes(in_ref[0, :, ec, :], 0, 1)


def _tc_transpose_part(part, b, buf, out_shape):
    one, seq, ecs, lanes = part.shape
    grid = (seq // _SBLK, ecs // _ECHB)
    in_specs = [
        pl.BlockSpec((1, _SBLK, _ECHB, lanes), lambda i, j: (0, i, j, 0)),
    ]
    kwargs = {}
    operands = (part,)
    if buf is not None:
        in_specs.append(pl.BlockSpec(memory_space=pl.ANY))
        operands = (part, buf)
        kwargs["input_output_aliases"] = {1: 0}
    return pl.pallas_call(
        _transpose_part_body,
        grid=grid,
        in_specs=in_specs,
        out_specs=pl.BlockSpec((1, _ECHB, lanes, _SBLK),
                               lambda i, j: (b, j, 0, i)),
        out_shape=out_shape,
        **kwargs,
    )(*operands)


def _tc_transpose(x):
    batch, seq, ecs, lanes = x.shape
    grid = (batch, seq // _SBLK, ecs // _ECHB)
    return pl.pallas_call(
        _transpose_body,
        grid=grid,
        in_specs=[pl.BlockSpec((1, _SBLK, _ECHB, lanes),
                               lambda b, i, j: (b, i, j, 0))],
        out_specs=pl.BlockSpec((1, _ECHB, lanes, _SBLK),
                               lambda b, i, j: (b, j, 0, i)),
        out_shape=jax.ShapeDtypeStruct((batch, ecs, lanes, seq),
                                       jnp.float32),
    )(x)


@jax.jit
def kernel(mapping, maps):
    batch, seq = mapping.shape
    tokens = batch * seq
    tokens_per_worker = tokens // _NW
    n_chunks = tokens_per_worker // _CHUNK

    m4 = jnp.transpose(maps, (1, 2, 3, 0)).reshape(_ROW // 128, 128,
                                                   maps.shape[0])
    table = _tc_transpose_in(m4)

    out_shape = jax.ShapeDtypeStruct((batch, _ROW // 128, 128, seq),
                                     jnp.float32)
    buf = None
    for b in range(batch):
        idx_b = mapping[b].reshape(_NW, seq // _NW // _CHUNK, _CHUNK)
        g = _sc_gather(table, idx_b, seq)
        g = g.reshape(1, seq, _ROW // 128, 128)
        buf = _tc_transpose_part(g, b, buf, out_shape)
    out = buf.reshape(batch, _HEADS, _DIM, _DIM, seq)
    return jnp.transpose(out, (0, 4, 1, 2, 3))
